# baseline (speedup 1.0000x reference)
import jax
import jax.numpy as jnp
from jax import lax
from jax.experimental import pallas as pl
from jax.experimental.pallas import tpu as pltpu

_T, _T2, _B, _D, _M = 2000, 1635, 1000, 35, 4
_T0 = 360
_TT = 8
_NBLK = (_T - _T0) // _TT
_OFFB = _T0 // _TT


def _ens_kernel(xc_ref, p0_ref, p1_ref, p2_ref, p3_ref, w_ref, b_ref, out_ref):
    x = xc_ref[...].reshape(_D * _TT, _B)
    logits = lax.dot_general(
        w_ref[...], x, (((0,), (0,)), ((), ())),
        preferred_element_type=jnp.float32)
    l0 = logits[0 * _TT:1 * _TT, :] + b_ref[0]
    l1 = logits[1 * _TT:2 * _TT, :] + b_ref[1]
    l2 = logits[2 * _TT:3 * _TT, :] + b_ref[2]
    l3 = logits[3 * _TT:4 * _TT, :] + b_ref[3]
    p01 = jnp.where(l0 >= l1, p0_ref[...], p1_ref[...])
    v01 = jnp.maximum(l0, l1)
    p23 = jnp.where(l2 >= l3, p2_ref[...], p3_ref[...])
    v23 = jnp.maximum(l2, l3)
    out_ref[...] = jnp.where(v01 >= v23, p01, p23)


def kernel(xc_nn_norm, target, pred_m0, pred_m1, pred_m2, pred_m3, W, b):
    del target
    xc_t = jnp.transpose(xc_nn_norm, (2, 0, 1))
    pad = (_T - _T0) - _T2
    pp = [
        jnp.pad(p.reshape(_T2, _B), ((pad, 0), (0, 0)))
        for p in (pred_m0, pred_m1, pred_m2, pred_m3)
    ]
    wbig = jnp.einsum('dm,jk->djmk', W, jnp.eye(_TT, dtype=W.dtype))
    wbig = wbig.reshape(_D * _TT, _M * _TT)
    pspec = pl.BlockSpec((_TT, _B), lambda i: (i, 0))
    out = pl.pallas_call(
        _ens_kernel,
        grid=(_NBLK,),
        in_specs=[
            pl.BlockSpec((_D, _TT, _B), lambda i: (0, i + _OFFB, 0)),
            pspec, pspec, pspec, pspec,
            pl.BlockSpec((_D * _TT, _M * _TT), lambda i: (0, 0)),
            pl.BlockSpec(memory_space=pltpu.SMEM),
        ],
        out_specs=pl.BlockSpec((_TT, _B), lambda i: (i, 0)),
        out_shape=jax.ShapeDtypeStruct((_T2 + pad, _B), jnp.float32),
    )(xc_t, *pp, wbig, b)
    return out[pad:]


# speedup vs baseline: 1.3446x; 1.3446x over previous
import jax
import jax.numpy as jnp
from jax import lax
from jax.experimental import pallas as pl
from jax.experimental.pallas import tpu as pltpu

_T, _T2, _B, _D, _M = 2000, 1635, 1000, 35, 4
_T0 = 360
_TT = 8
_NSUB = 5
_TB = _TT * _NSUB
_NBLK = (_T - _T0) // _TB
_OFFB = _T0 // _TB


def _ens_kernel(xc_ref, p0_ref, p1_ref, p2_ref, p3_ref, w_ref, b_ref, out_ref):
    for k in range(_NSUB):
        s = slice(k * _TT, (k + 1) * _TT)
        x = xc_ref[:, s, :].reshape(_D * _TT, _B)
        logits = lax.dot_general(
            w_ref[...], x, (((0,), (0,)), ((), ())),
            preferred_element_type=jnp.float32)
        l0 = logits[0 * _TT:1 * _TT, :] + b_ref[0]
        l1 = logits[1 * _TT:2 * _TT, :] + b_ref[1]
        l2 = logits[2 * _TT:3 * _TT, :] + b_ref[2]
        l3 = logits[3 * _TT:4 * _TT, :] + b_ref[3]
        p01 = jnp.where(l0 >= l1, p0_ref[s, :], p1_ref[s, :])
        v01 = jnp.maximum(l0, l1)
        p23 = jnp.where(l2 >= l3, p2_ref[s, :], p3_ref[s, :])
        v23 = jnp.maximum(l2, l3)
        out_ref[s, :] = jnp.where(v01 >= v23, p01, p23)


def kernel(xc_nn_norm, target, pred_m0, pred_m1, pred_m2, pred_m3, W, b):
    del target
    xc_t = jnp.transpose(xc_nn_norm, (2, 0, 1))
    pad = (_T - _T0) - _T2
    pp = [
        jnp.pad(p.reshape(_T2, _B), ((pad, 0), (0, 0)))
        for p in (pred_m0, pred_m1, pred_m2, pred_m3)
    ]
    wbig = jnp.einsum('dm,jk->djmk', W, jnp.eye(_TT, dtype=W.dtype))
    wbig = wbig.reshape(_D * _TT, _M * _TT)
    pspec = pl.BlockSpec((_TB, _B), lambda i: (i, 0))
    out = pl.pallas_call(
        _ens_kernel,
        grid=(_NBLK,),
        in_specs=[
            pl.BlockSpec((_D, _TB, _B), lambda i: (0, i + _OFFB, 0)),
            pspec, pspec, pspec, pspec,
            pl.BlockSpec((_D * _TT, _M * _TT), lambda i: (0, 0)),
            pl.BlockSpec(memory_space=pltpu.SMEM),
        ],
        out_specs=pl.BlockSpec((_TB, _B), lambda i: (i, 0)),
        out_shape=jax.ShapeDtypeStruct((_T2 + pad, _B), jnp.float32),
    )(xc_t, *pp, wbig, b)
    return out[pad:]


# speedup vs baseline: 3.0056x; 2.2354x over previous
import jax
import jax.numpy as jnp
from jax import lax
from jax.experimental import pallas as pl
from jax.experimental.pallas import tpu as pltpu

_T, _T2, _B, _D, _M = 2000, 1635, 1000, 35, 4
_T0 = 360
_PAD = (_T - _T0) - _T2
_TP = _T - _T0
_TT = 8
_NSUB = 5
_TB = _TT * _NSUB
_NBLK = _TP // _TB
_OFFB = _T0 // _TB


def _pred_kernel(p_ref, o_ref):
    v = p_ref[...].reshape(_B, _T2)
    o_ref[_PAD:_TP, :] = jnp.transpose(v, (1, 0))
    o_ref[0:_PAD, :] = jnp.zeros((_PAD, _B), jnp.float32)


def _ens_kernel(xc_ref, p0_ref, p1_ref, p2_ref, p3_ref, w_ref, b_ref, out_ref):
    for k in range(_NSUB):
        s = slice(k * _TT, (k + 1) * _TT)
        x = xc_ref[:, s, :].reshape(_D * _TT, _B)
        logits = lax.dot_general(
            w_ref[...], x, (((0,), (0,)), ((), ())),
            preferred_element_type=jnp.float32)
        l0 = logits[0 * _TT:1 * _TT, :] + b_ref[0]
        l1 = logits[1 * _TT:2 * _TT, :] + b_ref[1]
        l2 = logits[2 * _TT:3 * _TT, :] + b_ref[2]
        l3 = logits[3 * _TT:4 * _TT, :] + b_ref[3]
        p01 = jnp.where(l0 >= l1, p0_ref[s, :], p1_ref[s, :])
        v01 = jnp.maximum(l0, l1)
        p23 = jnp.where(l2 >= l3, p2_ref[s, :], p3_ref[s, :])
        v23 = jnp.maximum(l2, l3)
        out_ref[s, :] = jnp.where(v01 >= v23, p01, p23)


def _out_kernel(i_ref, o_ref):
    o_ref[...] = jnp.transpose(i_ref[_PAD:_TP, :], (1, 0))


def _pred_transpose(pred):
    pv = jnp.transpose(pred, (1, 2, 0))
    return pl.pallas_call(
        _pred_kernel,
        grid=(1,),
        in_specs=[pl.BlockSpec((_B, 1, _T2), lambda i: (0, 0, 0))],
        out_specs=pl.BlockSpec((_TP, _B), lambda i: (0, 0)),
        out_shape=jax.ShapeDtypeStruct((_TP, _B), jnp.float32),
    )(pv)


def kernel(xc_nn_norm, target, pred_m0, pred_m1, pred_m2, pred_m3, W, b):
    del target
    xc_t = jnp.transpose(xc_nn_norm, (2, 0, 1))
    pp = [_pred_transpose(p)
          for p in (pred_m0, pred_m1, pred_m2, pred_m3)]
    wbig = jnp.einsum('dm,jk->djmk', W, jnp.eye(_TT, dtype=W.dtype))
    wbig = wbig.reshape(_D * _TT, _M * _TT)
    pspec = pl.BlockSpec((_TB, _B), lambda i: (i, 0))
    out = pl.pallas_call(
        _ens_kernel,
        grid=(_NBLK,),
        in_specs=[
            pl.BlockSpec((_D, _TB, _B), lambda i: (0, i + _OFFB, 0)),
            pspec, pspec, pspec, pspec,
            pl.BlockSpec((_D * _TT, _M * _TT), lambda i: (0, 0)),
            pl.BlockSpec(memory_space=pltpu.SMEM),
        ],
        out_specs=pl.BlockSpec((_TB, _B), lambda i: (i, 0)),
        out_shape=jax.ShapeDtypeStruct((_TP, _B), jnp.float32),
    )(xc_t, *pp, wbig, b)
    out_bt = pl.pallas_call(
        _out_kernel,
        grid=(1,),
        in_specs=[pl.BlockSpec((_TP, _B), lambda i: (0, 0))],
        out_specs=pl.BlockSpec((_B, _T2), lambda i: (0, 0)),
        out_shape=jax.ShapeDtypeStruct((_B, _T2), jnp.float32),
    )(out)
    return jnp.transpose(out_bt, (1, 0))


# speedup vs baseline: 3.1900x; 1.0613x over previous
import jax
import jax.numpy as jnp
from jax import lax
from jax.experimental import pallas as pl
from jax.experimental.pallas import tpu as pltpu

_T, _T2, _B, _D, _M = 2000, 1635, 1000, 35, 4
_T0 = 360
_PAD = (_T - _T0) - _T2
_TP = _T - _T0
_TT = 8
_NSUB = 5
_TB = _TT * _NSUB
_NBLK = _TP // _TB
_OFFB = _T0 // _TB


def _pred_kernel(p0_ref, p1_ref, p2_ref, p3_ref,
                 o0_ref, o1_ref, o2_ref, o3_ref):
    for p_ref, o_ref in ((p0_ref, o0_ref), (p1_ref, o1_ref),
                         (p2_ref, o2_ref), (p3_ref, o3_ref)):
        v = p_ref[...].reshape(_B, _T2)
        o_ref[_PAD:_TP, :] = jnp.transpose(v, (1, 0))
        o_ref[0:_PAD, :] = jnp.zeros((_PAD, _B), jnp.float32)


def _ens_kernel(xc_ref, p0_ref, p1_ref, p2_ref, p3_ref, w_ref, b_ref, out_ref):
    for k in range(_NSUB):
        s = slice(k * _TT, (k + 1) * _TT)
        x = xc_ref[:, s, :].reshape(_D * _TT, _B)
        logits = lax.dot_general(
            w_ref[...], x, (((0,), (0,)), ((), ())),
            preferred_element_type=jnp.float32)
        l0 = logits[0 * _TT:1 * _TT, :] + b_ref[0]
        l1 = logits[1 * _TT:2 * _TT, :] + b_ref[1]
        l2 = logits[2 * _TT:3 * _TT, :] + b_ref[2]
        l3 = logits[3 * _TT:4 * _TT, :] + b_ref[3]
        p01 = jnp.where(l0 >= l1, p0_ref[s, :], p1_ref[s, :])
        v01 = jnp.maximum(l0, l1)
        p23 = jnp.where(l2 >= l3, p2_ref[s, :], p3_ref[s, :])
        v23 = jnp.maximum(l2, l3)
        out_ref[s, :] = jnp.where(v01 >= v23, p01, p23)


def _out_kernel(i_ref, o_ref):
    o_ref[...] = jnp.transpose(i_ref[_PAD:_TP, :], (1, 0))


def _pred_transpose(preds):
    pvs = [jnp.transpose(p, (1, 2, 0)) for p in preds]
    return pl.pallas_call(
        _pred_kernel,
        grid=(1,),
        in_specs=[pl.BlockSpec((_B, 1, _T2), lambda i: (0, 0, 0))] * 4,
        out_specs=[pl.BlockSpec((_TP, _B), lambda i: (0, 0))] * 4,
        out_shape=[jax.ShapeDtypeStruct((_TP, _B), jnp.float32)] * 4,
    )(*pvs)


def kernel(xc_nn_norm, target, pred_m0, pred_m1, pred_m2, pred_m3, W, b):
    del target
    xc_t = jnp.transpose(xc_nn_norm, (2, 0, 1))
    pp = _pred_transpose((pred_m0, pred_m1, pred_m2, pred_m3))
    wbig = jnp.einsum('dm,jk->djmk', W, jnp.eye(_TT, dtype=W.dtype))
    wbig = wbig.reshape(_D * _TT, _M * _TT)
    pspec = pl.BlockSpec((_TB, _B), lambda i: (i, 0))
    out = pl.pallas_call(
        _ens_kernel,
        grid=(_NBLK,),
        in_specs=[
            pl.BlockSpec((_D, _TB, _B), lambda i: (0, i + _OFFB, 0)),
            pspec, pspec, pspec, pspec,
            pl.BlockSpec((_D * _TT, _M * _TT), lambda i: (0, 0)),
            pl.BlockSpec(memory_space=pltpu.SMEM),
        ],
        out_specs=pl.BlockSpec((_TB, _B), lambda i: (i, 0)),
        out_shape=jax.ShapeDtypeStruct((_TP, _B), jnp.float32),
    )(xc_t, *pp, wbig, b)
    out_bt = pl.pallas_call(
        _out_kernel,
        grid=(1,),
        in_specs=[pl.BlockSpec((_TP, _B), lambda i: (0, 0))],
        out_specs=pl.BlockSpec((_B, _T2), lambda i: (0, 0)),
        out_shape=jax.ShapeDtypeStruct((_B, _T2), jnp.float32),
    )(out)
    return jnp.transpose(out_bt, (1, 0))
